# Initial kernel scaffold; baseline (speedup 1.0000x reference)
import jax
import jax.numpy as jnp
from jax.experimental import pallas as pl


def kernel(seq, embedding, p):
    raise NotImplementedError("write your pallas kernel here")


# speedup vs baseline: 1.5642x; 1.5642x over previous
import functools
import math

import jax
import jax.numpy as jnp
from jax import lax
from jax.experimental import pallas as pl
from jax.experimental.pallas import tpu as pltpu
from jax.experimental.pallas import tpu_sc as plsc

_D = 128
_S = 200
_B = 1024
_NC = 2
_NS = 16
_NW = _NC * _NS
_ROWS = _B * _S
_RPW = _ROWS // _NW
_CH = 40
_NCH = _RPW // _CH
_CPS = _S // _CH
_SCALE = math.sqrt(float(_D))

_mesh = plsc.VectorSubcoreMesh(core_axis_name="c", subcore_axis_name="s")


@functools.partial(
    pl.kernel,
    mesh=_mesh,
    out_type=jax.ShapeDtypeStruct((_ROWS, _D), jnp.float32),
    scratch_types=[
        pltpu.VMEM((_RPW,), jnp.int32),
        pltpu.VMEM((_S, _D), jnp.float32),
        pltpu.VMEM((_CH, _D), jnp.float32),
        pltpu.SemaphoreType.DMA,
    ],
)
def _sc_embed(idx_hbm, p_hbm, table_hbm, out_hbm, idx_v, p_v, buf, gsem):
    wid = lax.axis_index("s") * _NC + lax.axis_index("c")
    base = wid * _RPW
    pltpu.sync_copy(idx_hbm.at[pl.ds(base, _RPW)], idx_v)
    pltpu.sync_copy(p_hbm, p_v)

    def chunk_body(c, carry):
        pltpu.async_copy(
            table_hbm.at[idx_v.at[pl.ds(c * _CH, _CH)]], buf, gsem
        ).wait()
        pbase = (c % _CPS) * _CH

        def row_body(r, carry2):
            for cc in range(_D // 16):
                sl = pl.ds(cc * 16, 16)
                buf[r, sl] = buf[r, sl] * _SCALE + p_v[pbase + r, sl]
            return carry2

        lax.fori_loop(0, _CH, row_body, 0, unroll=False)
        pltpu.sync_copy(buf, out_hbm.at[pl.ds(base + c * _CH, _CH)])
        return carry

    lax.fori_loop(0, _NCH, chunk_body, 0, unroll=False)


def kernel(seq, embedding, p):
    idx = seq.reshape(-1).astype(jnp.int32)
    out = _sc_embed(idx, p[:_S], embedding)
    return out.reshape(_B, _S, _D)


# speedup vs baseline: 2.5676x; 1.6414x over previous
import functools
import math

import jax
import jax.numpy as jnp
from jax import lax
from jax.experimental import pallas as pl
from jax.experimental.pallas import tpu as pltpu
from jax.experimental.pallas import tpu_sc as plsc

_D = 128
_S = 200
_B = 1024
_NC = 2
_NS = 16
_NW = _NC * _NS
_ROWS = _B * _S
_RPW = _ROWS // _NW
_CH = 40
_NCH = _RPW // _CH
_CPS = _S // _CH
_SCALE = math.sqrt(float(_D))

_mesh = plsc.VectorSubcoreMesh(core_axis_name="c", subcore_axis_name="s")


_NB = 4


@functools.partial(
    pl.kernel,
    mesh=_mesh,
    out_type=jax.ShapeDtypeStruct((_ROWS, _D), jnp.float32),
    scratch_types=[
        pltpu.VMEM((_RPW,), jnp.int32),
        pltpu.VMEM((_S, _D), jnp.float32),
    ]
    + [pltpu.VMEM((_CH, _D), jnp.float32) for _ in range(_NB)]
    + [pltpu.SemaphoreType.DMA for _ in range(2 * _NB)],
)
def _sc_embed(idx_hbm, p_hbm, table_hbm, out_hbm, idx_v, p_v, *rest):
    bufs = rest[:_NB]
    gsems = rest[_NB:2 * _NB]
    osems = rest[2 * _NB:]
    wid = lax.axis_index("s") * _NC + lax.axis_index("c")
    base = wid * _RPW
    pltpu.sync_copy(idx_hbm.at[pl.ds(base, _RPW)], idx_v)
    pltpu.sync_copy(p_hbm, p_v)

    def gather_issue(j, b):
        pltpu.async_copy(
            table_hbm.at[idx_v.at[pl.ds(j * _CH, _CH)]], bufs[b], gsems[b]
        )

    for b in range(_NB - 1):
        gather_issue(b, b)

    def outer(o, carry):
        for b in range(_NB):
            j = o * _NB + b
            pltpu.make_async_copy(
                table_hbm.at[idx_v.at[pl.ds(0, _CH)]], bufs[b], gsems[b]
            ).wait()
            pbase = (j % _CPS) * _CH

            def row_body(r, c2, _b=b, _pb=pbase):
                for cc in range(_D // 16):
                    sl = pl.ds(cc * 16, 16)
                    bufs[_b][r, sl] = bufs[_b][r, sl] * _SCALE + p_v[_pb + r, sl]
                return c2

            lax.fori_loop(0, _CH, row_body, 0, unroll=False)
            pltpu.async_copy(
                bufs[b], out_hbm.at[pl.ds(base + j * _CH, _CH)], osems[b]
            )
            nb = (b + _NB - 1) % _NB
            g = j + _NB - 1

            @pl.when(g < _NCH)
            def _():
                @pl.when(j >= 1)
                def _():
                    pltpu.make_async_copy(
                        bufs[nb], out_hbm.at[pl.ds(0, _CH)], osems[nb]
                    ).wait()

                gather_issue(g, nb)

        return carry

    lax.fori_loop(0, _NCH // _NB, outer, 0, unroll=False)
    for b in range(_NB):
        pltpu.make_async_copy(
            bufs[b], out_hbm.at[pl.ds(0, _CH)], osems[b]
        ).wait()


def kernel(seq, embedding, p):
    idx = seq.reshape(-1).astype(jnp.int32)
    out = _sc_embed(idx, p[:_S], embedding)
    return out.reshape(_B, _S, _D)
